# Initial kernel scaffold; baseline (speedup 1.0000x reference)
import jax
import jax.numpy as jnp
from jax.experimental import pallas as pl


def kernel(inputs, task_indices, W1, b1, W2, b2, W3, b3, headW, headb):
    raise NotImplementedError("write your pallas kernel here")


# speedup vs baseline: 1.7118x; 1.7118x over previous
import jax
import jax.numpy as jnp
from jax.experimental import pallas as pl

_N = 32768
_D = 768
_H = 64
_T = 64
_O = 64
_BLK = 256


def _trunk_head_kernel(x_ref, ti_ref, W1_ref, b1_ref, W2_ref, b2_ref,
                       W3_ref, b3_ref, hWr_ref, hb_ref, out_ref):
    x = x_ref[...]
    h = jnp.tanh(jnp.dot(x, W1_ref[...], preferred_element_type=jnp.float32)
                 + b1_ref[...])
    h = jnp.tanh(jnp.dot(h, W2_ref[...], preferred_element_type=jnp.float32)
                 + b2_ref[...])
    h = jnp.tanh(jnp.dot(h, W3_ref[...], preferred_element_type=jnp.float32)
                 + b3_ref[...])
    ti = ti_ref[0, 0, :].reshape(_BLK, 1)
    iota_t = jax.lax.broadcasted_iota(jnp.int32, (_BLK, _T), 1)
    onehot = (iota_t == ti).astype(jnp.float32)
    B = (onehot[:, :, None] * h[:, None, :]).reshape(_BLK, _T * _H)
    out = (jnp.dot(B, hWr_ref[...], preferred_element_type=jnp.float32)
           + jnp.dot(onehot, hb_ref[...], preferred_element_type=jnp.float32))
    out_ref[...] = out


def kernel(inputs, task_indices, W1, b1, W2, b2, W3, b3, headW, headb):
    n_blocks = _N // _BLK
    ti3 = task_indices.astype(jnp.int32).reshape(n_blocks, 1, _BLK)
    hWr = headW.reshape(_T * _H, _O)
    b1r = b1.reshape(1, _H)
    b2r = b2.reshape(1, _H)
    b3r = b3.reshape(1, _H)

    grid = (n_blocks,)
    out = pl.pallas_call(
        _trunk_head_kernel,
        grid=grid,
        in_specs=[
            pl.BlockSpec((_BLK, _D), lambda i: (i, 0)),
            pl.BlockSpec((1, 1, _BLK), lambda i: (i, 0, 0)),
            pl.BlockSpec((_D, _H), lambda i: (0, 0)),
            pl.BlockSpec((1, _H), lambda i: (0, 0)),
            pl.BlockSpec((_H, _H), lambda i: (0, 0)),
            pl.BlockSpec((1, _H), lambda i: (0, 0)),
            pl.BlockSpec((_H, _H), lambda i: (0, 0)),
            pl.BlockSpec((1, _H), lambda i: (0, 0)),
            pl.BlockSpec((_T * _H, _O), lambda i: (0, 0)),
            pl.BlockSpec((_T, _O), lambda i: (0, 0)),
        ],
        out_specs=pl.BlockSpec((_BLK, _O), lambda i: (i, 0)),
        out_shape=jax.ShapeDtypeStruct((_N, _O), jnp.float32),
    )(inputs, ti3, W1, b1r, W2, b2r, W3, b3r, hWr, headb)
    return out


# speedup vs baseline: 4.2687x; 2.4937x over previous
import jax
import jax.numpy as jnp
from jax.experimental import pallas as pl

_N = 32768
_D = 768
_H = 64
_T = 64
_O = 64
_BLK = 256


def _trunk_head_kernel(x_ref, ti_ref, W1_ref, b1_ref, W2_ref, b2_ref,
                       W3_ref, b3_ref, W2d_ref, hbf_ref, out_ref):
    x = x_ref[...]
    h = jnp.tanh(jnp.dot(x, W1_ref[...], preferred_element_type=jnp.float32)
                 + b1_ref[...])
    h = jnp.tanh(jnp.dot(h, W2_ref[...], preferred_element_type=jnp.float32)
                 + b2_ref[...])
    h = jnp.tanh(jnp.dot(h, W3_ref[...], preferred_element_type=jnp.float32)
                 + b3_ref[...])
    z = jnp.dot(h, W2d_ref[...], preferred_element_type=jnp.float32)
    z = z + hbf_ref[...]
    ti = ti_ref[0, 0, :].reshape(_BLK, 1)
    col_task = jax.lax.broadcasted_iota(jnp.int32, (_BLK, _T * _O), 1) // _O
    z = jnp.where(col_task == ti, z, 0.0)
    w = (_T * _O) // 2
    while w >= _O:
        z = z[:, :w] + z[:, w:]
        w //= 2
    out_ref[...] = z


def kernel(inputs, task_indices, W1, b1, W2, b2, W3, b3, headW, headb):
    n_blocks = _N // _BLK
    ti3 = task_indices.astype(jnp.int32).reshape(n_blocks, 1, _BLK)
    W2d = headW.transpose(1, 0, 2).reshape(_H, _T * _O)
    hbf = headb.reshape(1, _T * _O)
    b1r = b1.reshape(1, _H)
    b2r = b2.reshape(1, _H)
    b3r = b3.reshape(1, _H)

    grid = (n_blocks,)
    out = pl.pallas_call(
        _trunk_head_kernel,
        grid=grid,
        in_specs=[
            pl.BlockSpec((_BLK, _D), lambda i: (i, 0)),
            pl.BlockSpec((1, 1, _BLK), lambda i: (i, 0, 0)),
            pl.BlockSpec((_D, _H), lambda i: (0, 0)),
            pl.BlockSpec((1, _H), lambda i: (0, 0)),
            pl.BlockSpec((_H, _H), lambda i: (0, 0)),
            pl.BlockSpec((1, _H), lambda i: (0, 0)),
            pl.BlockSpec((_H, _H), lambda i: (0, 0)),
            pl.BlockSpec((1, _H), lambda i: (0, 0)),
            pl.BlockSpec((_H, _T * _O), lambda i: (0, 0)),
            pl.BlockSpec((1, _T * _O), lambda i: (0, 0)),
        ],
        out_specs=pl.BlockSpec((_BLK, _O), lambda i: (i, 0)),
        out_shape=jax.ShapeDtypeStruct((_N, _O), jnp.float32),
    )(inputs, ti3, W1, b1r, W2, b2r, W3, b3r, W2d, hbf)
    return out
